# baseline (speedup 1.0000x reference)
import functools

import jax
import jax.numpy as jnp
from jax import lax
from jax.experimental import pallas as pl
from jax.experimental.pallas import tpu as pltpu
from jax.experimental.pallas import tpu_sc as plsc

N = 10000
E = 320000
D = 128
CLS = 64
NC = 2
NS = 16
NW = NC * NS
CHUNK_A = 104
CHUNK_B = 96
PAIR = CHUNK_A + CHUNK_B
N_PAIRS = 50
E_PER_W = PAIR * N_PAIRS
CCHUNK = 128
CN_CHUNKS = 80
E_PAD = CCHUNK * CN_CHUNKS * NW
N_PAD = 10240
ROWS_PER_TILE = N_PAD // NS
ZROWS = 40
CZROWS = 160
CW = 128


_MESH = plsc.VectorSubcoreMesh(core_axis_name="c", subcore_axis_name="s")


def _build_agg(interpret=False):
  @functools.partial(
    pl.kernel, mesh=_MESH, interpret=interpret,
    out_type=jax.ShapeDtypeStruct((NC, N_PAD, D), jnp.float32),
    scratch_types=[
        pltpu.VMEM((CHUNK_A,), jnp.int32),
        pltpu.VMEM((CHUNK_A,), jnp.int32),
        pltpu.VMEM((CHUNK_B,), jnp.int32),
        pltpu.VMEM((CHUNK_B,), jnp.int32),
        pltpu.VMEM((CHUNK_A, D), jnp.float32),
        pltpu.VMEM((CHUNK_B, D), jnp.float32),
        pltpu.VMEM_SHARED((N_PAD, D), jnp.float32),
        pltpu.SemaphoreType.DMA,
        pltpu.SemaphoreType.DMA,
        pltpu.SemaphoreType.DMA,
        pltpu.SemaphoreType.DMA,
    ])
  def _agg(table, src, dst, zrows, out,
           src_v0, dst_v0, src_v1, dst_v1, rows_v0, rows_v1, acc,
           gsem0, gsem1, ssem0, ssem1):
    cid = lax.axis_index("c")
    sid = lax.axis_index("s")
    wid = sid * NC + cid

    row0 = sid * ROWS_PER_TILE
    pltpu.sync_copy(zrows, acc.at[pl.ds(row0, ROWS_PER_TILE)])
    plsc.subcore_barrier()

    base = wid * E_PER_W
    ib = ((src_v0, dst_v0), (src_v1, dst_v1))
    rows = (rows_v0, rows_v1)
    sizes = (CHUNK_A, CHUNK_B)
    gsems = (gsem0, gsem1)
    ssems = (ssem0, ssem1)

    def load_idx(k, b):
        sv, dv = ib[b]
        off = base + k * PAIR + (CHUNK_A if b else 0)
        pltpu.sync_copy(src.at[pl.ds(off, sizes[b])], sv)
        pltpu.sync_copy(dst.at[pl.ds(off, sizes[b])], dv)

    def gather_start(b):
        pltpu.async_copy(table.at[ib[b][0]], rows[b], gsems[b])

    def gather_wait(b):
        pltpu.make_async_copy(table.at[ib[b][0]], rows[b], gsems[b]).wait()

    def scatter_start(b):
        pltpu.async_copy(rows[b], acc.at[ib[b][1]], ssems[b], add=True)

    def scatter_wait(b):
        pltpu.make_async_copy(rows[b], acc.at[ib[b][1]], ssems[b]).wait()

    load_idx(0, 0)
    gather_start(0)
    gather_wait(0)
    scatter_start(0)
    load_idx(0, 1)
    gather_start(1)

    def pair(i, _):
        gather_wait(1)
        scatter_wait(0)
        scatter_start(1)
        load_idx(i + 1, 0)
        gather_start(0)
        gather_wait(0)
        scatter_wait(1)
        scatter_start(0)
        load_idx(i + 1, 1)
        gather_start(1)
        return 0
    lax.fori_loop(0, N_PAIRS - 1, pair, 0)

    gather_wait(1)
    scatter_wait(0)
    scatter_start(1)
    scatter_wait(1)

    plsc.subcore_barrier()
    pltpu.sync_copy(acc.at[pl.ds(row0, ROWS_PER_TILE)],
                    out.at[cid, pl.ds(row0, ROWS_PER_TILE)])

  return _agg


def _build_cnt(interpret=False):
  @functools.partial(
    pl.kernel, mesh=_MESH, interpret=interpret,
    out_type=jax.ShapeDtypeStruct((NC, N_PAD, CW), jnp.float32),
    scratch_types=[
        pltpu.VMEM((CN_CHUNKS, CCHUNK), jnp.int32),
        pltpu.VMEM((CCHUNK, CW), jnp.float32),
        pltpu.VMEM_SHARED((N_PAD, CW), jnp.float32),
    ])
  def _cnt(dst, ones, zrows, cnt_out, dst_blk, ones_v, cacc):
    cid = lax.axis_index("c")
    sid = lax.axis_index("s")
    wid = sid * NC + cid

    pltpu.sync_copy(ones, ones_v)
    row0 = sid * ROWS_PER_TILE
    pltpu.sync_copy(zrows, cacc.at[pl.ds(row0, ROWS_PER_TILE)])
    pltpu.sync_copy(dst.at[pl.ds(wid * CN_CHUNKS, CN_CHUNKS)], dst_blk)
    plsc.subcore_barrier()

    def step(j, _):
        pltpu.sync_copy(ones_v, cacc.at[dst_blk.at[j]], add=True)
        return 0
    lax.fori_loop(0, CN_CHUNKS, step, 0)

    plsc.subcore_barrier()
    pltpu.sync_copy(cacc.at[pl.ds(row0, ROWS_PER_TILE)],
                    cnt_out.at[cid, pl.ds(row0, ROWS_PER_TILE)])

  return _cnt


_agg = _build_agg()
_cnt = _build_cnt()

RB = 2000


def _sage_body(p0, p1, c0, c1, h, wl, wr, b, o):
    cnt = jnp.maximum(c0[:, 0:1] + c1[:, 0:1], 1.0)
    mean = (p0[...] + p1[...]) / cnt
    acc = jnp.dot(mean, wl[...], preferred_element_type=jnp.float32)
    acc += jnp.dot(h[...], wr[...], preferred_element_type=jnp.float32)
    o[...] = jnp.maximum(acc + b[...], 0.0)


def _final_body(p0, p1, c0, c1, h, wl, wr, b, wc, bc, o):
    cnt = jnp.maximum(c0[:, 0:1] + c1[:, 0:1], 1.0)
    mean = (p0[...] + p1[...]) / cnt
    acc = jnp.dot(mean, wl[...], preferred_element_type=jnp.float32)
    acc += jnp.dot(h[...], wr[...], preferred_element_type=jnp.float32)
    h2 = jnp.maximum(acc + b[...], 0.0)
    o[...] = jnp.dot(h2, wc[...], preferred_element_type=jnp.float32) + bc[...]


def _row_spec(w):
    return pl.BlockSpec((RB, w), lambda i: (i, 0))


def _full_spec(r, c):
    return pl.BlockSpec((r, c), lambda i: (0, 0))


def _sage_tc(p0, p1, c0, c1, h, wl, wr, b):
    return pl.pallas_call(
        _sage_body,
        grid=(N // RB,),
        in_specs=[_row_spec(D), _row_spec(D), _row_spec(CW), _row_spec(CW),
                  _row_spec(D), _full_spec(D, D), _full_spec(D, D),
                  _full_spec(1, D)],
        out_specs=_row_spec(D),
        out_shape=jax.ShapeDtypeStruct((N, D), jnp.float32),
    )(p0, p1, c0, c1, h, wl, wr, b)


def _final_tc(p0, p1, c0, c1, h, wl, wr, b, wc, bc):
    return pl.pallas_call(
        _final_body,
        grid=(N // RB,),
        in_specs=[_row_spec(D), _row_spec(D), _row_spec(CW), _row_spec(CW),
                  _row_spec(D), _full_spec(D, D), _full_spec(D, D),
                  _full_spec(1, D), _full_spec(D, CLS), _full_spec(1, CLS)],
        out_specs=_row_spec(CLS),
        out_shape=jax.ShapeDtypeStruct((N, CLS), jnp.float32),
    )(p0, p1, c0, c1, h, wl, wr, b, wc, bc)


def kernel(x, edge_index, W1_l, b1, W1_r, W2_l, b2, W2_r, Wc, bc):
    ei = edge_index.astype(jnp.int32)
    src, dst = ei[0], ei[1]
    pad = E_PAD - E
    pad_dst = N + (jnp.arange(pad, dtype=jnp.int32) % (N_PAD - N))
    dst2 = jnp.concatenate([dst, pad_dst]).reshape(NW * CN_CHUNKS, CCHUNK)
    zrows = jnp.zeros((ROWS_PER_TILE, D), jnp.float32)
    ones = jnp.ones((CCHUNK, CW), jnp.float32)
    cnts = _cnt(dst2, ones, zrows)
    parts1 = _agg(x, src, dst, zrows)
    c0, c1 = cnts[0], cnts[1]
    h1 = _sage_tc(parts1[0], parts1[1], c0, c1, x,
                  W1_l, W1_r, b1.reshape(1, D))
    parts2 = _agg(h1, src, dst, zrows)
    return _final_tc(parts2[0], parts2[1], c0, c1, h1,
                     W2_l, W2_r, b2.reshape(1, D), Wc, bc.reshape(1, CLS))


# speedup vs baseline: 1.2874x; 1.2874x over previous
import functools

import jax
import jax.numpy as jnp
from jax import lax
from jax.experimental import pallas as pl
from jax.experimental.pallas import tpu as pltpu
from jax.experimental.pallas import tpu_sc as plsc

N = 10000
E = 320000
D = 128
CLS = 64
NC = 2
NS = 16
NW = NC * NS
CHUNK = 200
N_CHUNKS = 50
E_PER_W = CHUNK * N_CHUNKS
CCHUNK = 128
CN_CHUNKS = 80
E_PAD = CCHUNK * CN_CHUNKS * NW
N_PAD = 10240
ROWS_PER_TILE = N_PAD // NS
ZROWS = 40
CZROWS = 160
CW = 128


_MESH = plsc.VectorSubcoreMesh(core_axis_name="c", subcore_axis_name="s")


def _build_agg(interpret=False):
  @functools.partial(
    pl.kernel, mesh=_MESH, interpret=interpret,
    out_type=jax.ShapeDtypeStruct((NC, N_PAD, D), jnp.float32),
    scratch_types=[
        pltpu.VMEM((CHUNK,), jnp.int32),
        pltpu.VMEM((CHUNK,), jnp.int32),
        pltpu.VMEM((CHUNK,), jnp.int32),
        pltpu.VMEM((CHUNK,), jnp.int32),
        pltpu.VMEM((CHUNK, D), jnp.float32),
        pltpu.VMEM_SHARED((N_PAD, D), jnp.float32),
        pltpu.SemaphoreType.DMA,
    ])
  def _agg(table, src, dst, zrows, out,
           src_v0, dst_v0, src_v1, dst_v1, rows_v, acc, sem):
    cid = lax.axis_index("c")
    sid = lax.axis_index("s")
    wid = sid * NC + cid

    row0 = sid * ROWS_PER_TILE
    pltpu.sync_copy(zrows, acc.at[pl.ds(row0, ROWS_PER_TILE)])
    plsc.subcore_barrier()

    base = wid * E_PER_W
    ib = ((src_v0, dst_v0), (src_v1, dst_v1))

    def load_idx(g, b):
        sv, dv = ib[b]
        off = base + g * CHUNK
        pltpu.sync_copy(src.at[pl.ds(off, CHUNK)], sv)
        pltpu.sync_copy(dst.at[pl.ds(off, CHUNK)], dv)

    def work(b, prefetch):
        h = pltpu.async_copy(table.at[ib[b][0]], rows_v, sem)
        if prefetch is not None:
            load_idx(*prefetch)
        h.wait()
        pltpu.sync_copy(rows_v, acc.at[ib[b][1]], add=True)

    load_idx(0, 0)

    def pair(i2, _):
        g = 2 * i2
        work(0, (g + 1, 1))
        work(1, (g + 2, 0))
        return 0
    lax.fori_loop(0, N_CHUNKS // 2 - 1, pair, 0)
    work(0, (N_CHUNKS - 1, 1))
    work(1, None)

    plsc.subcore_barrier()
    pltpu.sync_copy(acc.at[pl.ds(row0, ROWS_PER_TILE)],
                    out.at[cid, pl.ds(row0, ROWS_PER_TILE)])

  return _agg


def _build_cnt(interpret=False):
  @functools.partial(
    pl.kernel, mesh=_MESH, interpret=interpret,
    out_type=jax.ShapeDtypeStruct((NC, N_PAD, CW), jnp.float32),
    scratch_types=[
        pltpu.VMEM((CN_CHUNKS, CCHUNK), jnp.int32),
        pltpu.VMEM((CCHUNK, CW), jnp.float32),
        pltpu.VMEM_SHARED((N_PAD, CW), jnp.float32),
    ])
  def _cnt(dst, ones, zrows, cnt_out, dst_blk, ones_v, cacc):
    cid = lax.axis_index("c")
    sid = lax.axis_index("s")
    wid = sid * NC + cid

    pltpu.sync_copy(ones, ones_v)
    row0 = sid * ROWS_PER_TILE
    pltpu.sync_copy(zrows, cacc.at[pl.ds(row0, ROWS_PER_TILE)])
    pltpu.sync_copy(dst.at[pl.ds(wid * CN_CHUNKS, CN_CHUNKS)], dst_blk)
    plsc.subcore_barrier()

    def step(j, _):
        pltpu.sync_copy(ones_v, cacc.at[dst_blk.at[j]], add=True)
        return 0
    lax.fori_loop(0, CN_CHUNKS, step, 0)

    plsc.subcore_barrier()
    pltpu.sync_copy(cacc.at[pl.ds(row0, ROWS_PER_TILE)],
                    cnt_out.at[cid, pl.ds(row0, ROWS_PER_TILE)])

  return _cnt


_agg = _build_agg()
_cnt = _build_cnt()

RB = 2000


def _sage_body(p0, p1, c0, c1, h, wl, wr, b, o):
    cnt = jnp.maximum(c0[:, 0:1] + c1[:, 0:1], 1.0)
    mean = (p0[...] + p1[...]) / cnt
    acc = jnp.dot(mean, wl[...], preferred_element_type=jnp.float32)
    acc += jnp.dot(h[...], wr[...], preferred_element_type=jnp.float32)
    o[...] = jnp.maximum(acc + b[...], 0.0)


def _final_body(p0, p1, c0, c1, h, wl, wr, b, wc, bc, o):
    cnt = jnp.maximum(c0[:, 0:1] + c1[:, 0:1], 1.0)
    mean = (p0[...] + p1[...]) / cnt
    acc = jnp.dot(mean, wl[...], preferred_element_type=jnp.float32)
    acc += jnp.dot(h[...], wr[...], preferred_element_type=jnp.float32)
    h2 = jnp.maximum(acc + b[...], 0.0)
    o[...] = jnp.dot(h2, wc[...], preferred_element_type=jnp.float32) + bc[...]


def _row_spec(w):
    return pl.BlockSpec((RB, w), lambda i: (i, 0))


def _full_spec(r, c):
    return pl.BlockSpec((r, c), lambda i: (0, 0))


def _sage_tc(p0, p1, c0, c1, h, wl, wr, b):
    return pl.pallas_call(
        _sage_body,
        grid=(N // RB,),
        in_specs=[_row_spec(D), _row_spec(D), _row_spec(CW), _row_spec(CW),
                  _row_spec(D), _full_spec(D, D), _full_spec(D, D),
                  _full_spec(1, D)],
        out_specs=_row_spec(D),
        out_shape=jax.ShapeDtypeStruct((N, D), jnp.float32),
    )(p0, p1, c0, c1, h, wl, wr, b)


def _final_tc(p0, p1, c0, c1, h, wl, wr, b, wc, bc):
    return pl.pallas_call(
        _final_body,
        grid=(N // RB,),
        in_specs=[_row_spec(D), _row_spec(D), _row_spec(CW), _row_spec(CW),
                  _row_spec(D), _full_spec(D, D), _full_spec(D, D),
                  _full_spec(1, D), _full_spec(D, CLS), _full_spec(1, CLS)],
        out_specs=_row_spec(CLS),
        out_shape=jax.ShapeDtypeStruct((N, CLS), jnp.float32),
    )(p0, p1, c0, c1, h, wl, wr, b, wc, bc)


def kernel(x, edge_index, W1_l, b1, W1_r, W2_l, b2, W2_r, Wc, bc):
    ei = edge_index.astype(jnp.int32)
    src, dst = ei[0], ei[1]
    pad = E_PAD - E
    pad_dst = N + (jnp.arange(pad, dtype=jnp.int32) % (N_PAD - N))
    dst2 = jnp.concatenate([dst, pad_dst]).reshape(NW * CN_CHUNKS, CCHUNK)
    zrows = jnp.zeros((ROWS_PER_TILE, D), jnp.float32)
    ones = jnp.ones((CCHUNK, CW), jnp.float32)
    cnts = _cnt(dst2, ones, zrows)
    parts1 = _agg(x, src, dst, zrows)
    c0, c1 = cnts[0], cnts[1]
    h1 = _sage_tc(parts1[0], parts1[1], c0, c1, x,
                  W1_l, W1_r, b1.reshape(1, D))
    parts2 = _agg(h1, src, dst, zrows)
    return _final_tc(parts2[0], parts2[1], c0, c1, h1,
                     W2_l, W2_r, b2.reshape(1, D), Wc, bc.reshape(1, CLS))


# speedup vs baseline: 1.4138x; 1.0982x over previous
import functools

import jax
import jax.numpy as jnp
from jax import lax
from jax.experimental import pallas as pl
from jax.experimental.pallas import tpu as pltpu
from jax.experimental.pallas import tpu_sc as plsc

N = 10000
E = 320000
D = 128
CLS = 64
NC = 2
NS = 16
NW = NC * NS
CHUNK = 128
N_CHUNKS = 80
HALF = N_CHUNKS // 2
E_PER_W = CHUNK * N_CHUNKS
E_PAD = E_PER_W * NW
PAD_PER_W = 240
N_PAD = 10496
SPARE_PER_W = 15
ROWS_PER_TILE = N_PAD // NS
ZROWS = 40
CZROWS = 160
CW = 128


_MESH = plsc.VectorSubcoreMesh(core_axis_name="c", subcore_axis_name="s")


def _build_agg(interpret=False):
  @functools.partial(
    pl.kernel, mesh=_MESH, interpret=interpret,
    out_type=jax.ShapeDtypeStruct((NC, N_PAD, D), jnp.float32),
    scratch_types=[
        pltpu.VMEM((HALF, CHUNK), jnp.int32),
        pltpu.VMEM((HALF, CHUNK), jnp.int32),
        pltpu.VMEM((CHUNK, D), jnp.float32),
        pltpu.VMEM((CHUNK, D), jnp.float32),
        pltpu.VMEM_SHARED((N_PAD, D), jnp.float32),
        pltpu.SemaphoreType.DMA,
        pltpu.SemaphoreType.DMA,
        pltpu.SemaphoreType.DMA,
        pltpu.SemaphoreType.DMA,
    ])
  def _agg(table, src, dst, zrows, out,
           src_blk, dst_blk, rows_v0, rows_v1, acc,
           gsem0, gsem1, ssem0, ssem1):
    cid = lax.axis_index("c")
    sid = lax.axis_index("s")
    wid = sid * NC + cid

    row0 = sid * ROWS_PER_TILE
    pltpu.sync_copy(zrows, acc.at[pl.ds(row0, ROWS_PER_TILE)])
    plsc.subcore_barrier()

    rows = (rows_v0, rows_v1)
    gsems = (gsem0, gsem1)
    ssems = (ssem0, ssem1)

    def gather_start(j, b):
        pltpu.async_copy(table.at[src_blk.at[j]], rows[b], gsems[b])

    def gather_wait(j, b):
        pltpu.make_async_copy(table.at[src_blk.at[j]], rows[b], gsems[b]).wait()

    def scatter_start(j, b):
        pltpu.async_copy(rows[b], acc.at[dst_blk.at[j]], ssems[b], add=True)

    def scatter_wait(j, b):
        pltpu.make_async_copy(rows[b], acc.at[dst_blk.at[j]], ssems[b]).wait()

    for h in range(N_CHUNKS // HALF):
        blk0 = wid * N_CHUNKS + h * HALF
        pltpu.sync_copy(src.at[pl.ds(blk0, HALF)], src_blk)
        pltpu.sync_copy(dst.at[pl.ds(blk0, HALF)], dst_blk)

        gather_start(0, 0)
        gather_wait(0, 0)
        scatter_start(0, 0)
        gather_start(1, 1)

        def pair(i, _):
            j = 2 * i + 1
            gather_wait(j, 1)
            scatter_wait(j - 1, 0)
            scatter_start(j, 1)
            gather_start(j + 1, 0)
            gather_wait(j + 1, 0)
            scatter_wait(j, 1)
            scatter_start(j + 1, 0)
            gather_start(j + 2, 1)
            return 0
        lax.fori_loop(0, HALF // 2 - 1, pair, 0)

        gather_wait(HALF - 1, 1)
        scatter_wait(HALF - 2, 0)
        scatter_start(HALF - 1, 1)
        scatter_wait(HALF - 1, 1)

    plsc.subcore_barrier()
    pltpu.sync_copy(acc.at[pl.ds(row0, ROWS_PER_TILE)],
                    out.at[cid, pl.ds(row0, ROWS_PER_TILE)])

  return _agg


def _build_cnt(interpret=False):
  @functools.partial(
    pl.kernel, mesh=_MESH, interpret=interpret,
    out_type=jax.ShapeDtypeStruct((NC, N_PAD, CW), jnp.float32),
    scratch_types=[
        pltpu.VMEM((N_CHUNKS, CHUNK), jnp.int32),
        pltpu.VMEM((CHUNK, CW), jnp.float32),
        pltpu.VMEM_SHARED((N_PAD, CW), jnp.float32),
    ])
  def _cnt(dst, ones, zrows, cnt_out, dst_blk, ones_v, cacc):
    cid = lax.axis_index("c")
    sid = lax.axis_index("s")
    wid = sid * NC + cid

    pltpu.sync_copy(ones, ones_v)
    row0 = sid * ROWS_PER_TILE
    pltpu.sync_copy(zrows, cacc.at[pl.ds(row0, ROWS_PER_TILE)])
    pltpu.sync_copy(dst.at[pl.ds(wid * N_CHUNKS, N_CHUNKS)], dst_blk)
    plsc.subcore_barrier()

    def step(j, _):
        pltpu.sync_copy(ones_v, cacc.at[dst_blk.at[j]], add=True)
        return 0
    lax.fori_loop(0, N_CHUNKS, step, 0)

    plsc.subcore_barrier()
    pltpu.sync_copy(cacc.at[pl.ds(row0, ROWS_PER_TILE)],
                    cnt_out.at[cid, pl.ds(row0, ROWS_PER_TILE)])

  return _cnt


_agg = _build_agg()
_cnt = _build_cnt()

RB = 2000


def _sage_body(p0, p1, c0, c1, h, wl, wr, b, o):
    cnt = jnp.maximum(c0[:, 0:1] + c1[:, 0:1], 1.0)
    mean = (p0[...] + p1[...]) / cnt
    acc = jnp.dot(mean, wl[...], preferred_element_type=jnp.float32)
    acc += jnp.dot(h[...], wr[...], preferred_element_type=jnp.float32)
    o[...] = jnp.maximum(acc + b[...], 0.0)


def _final_body(p0, p1, c0, c1, h, wl, wr, b, wc, bc, o):
    cnt = jnp.maximum(c0[:, 0:1] + c1[:, 0:1], 1.0)
    mean = (p0[...] + p1[...]) / cnt
    acc = jnp.dot(mean, wl[...], preferred_element_type=jnp.float32)
    acc += jnp.dot(h[...], wr[...], preferred_element_type=jnp.float32)
    h2 = jnp.maximum(acc + b[...], 0.0)
    o[...] = jnp.dot(h2, wc[...], preferred_element_type=jnp.float32) + bc[...]


def _row_spec(w):
    return pl.BlockSpec((RB, w), lambda i: (i, 0))


def _full_spec(r, c):
    return pl.BlockSpec((r, c), lambda i: (0, 0))


def _sage_tc(p0, p1, c0, c1, h, wl, wr, b):
    return pl.pallas_call(
        _sage_body,
        grid=(N // RB,),
        in_specs=[_row_spec(D), _row_spec(D), _row_spec(CW), _row_spec(CW),
                  _row_spec(D), _full_spec(D, D), _full_spec(D, D),
                  _full_spec(1, D)],
        out_specs=_row_spec(D),
        out_shape=jax.ShapeDtypeStruct((N, D), jnp.float32),
    )(p0, p1, c0, c1, h, wl, wr, b)


def _final_tc(p0, p1, c0, c1, h, wl, wr, b, wc, bc):
    return pl.pallas_call(
        _final_body,
        grid=(N // RB,),
        in_specs=[_row_spec(D), _row_spec(D), _row_spec(CW), _row_spec(CW),
                  _row_spec(D), _full_spec(D, D), _full_spec(D, D),
                  _full_spec(1, D), _full_spec(D, CLS), _full_spec(1, CLS)],
        out_specs=_row_spec(CLS),
        out_shape=jax.ShapeDtypeStruct((N, CLS), jnp.float32),
    )(p0, p1, c0, c1, h, wl, wr, b, wc, bc)


def kernel(x, edge_index, W1_l, b1, W1_r, W2_l, b2, W2_r, Wc, bc):
    ei = edge_index.astype(jnp.int32)
    w = jnp.arange(NW, dtype=jnp.int32)[:, None]
    i = jnp.arange(PAD_PER_W, dtype=jnp.int32)[None, :]
    pad_src = (w * 320 + i) % N
    pad_dst = N + w * SPARE_PER_W + (i % SPARE_PER_W)
    src = jnp.concatenate(
        [ei[0].reshape(NW, E // NW), pad_src], axis=1).reshape(
            NW * N_CHUNKS, CHUNK)
    dst = jnp.concatenate(
        [ei[1].reshape(NW, E // NW), pad_dst], axis=1).reshape(
            NW * N_CHUNKS, CHUNK)
    zrows = jnp.zeros((ROWS_PER_TILE, D), jnp.float32)
    ones = jnp.ones((CHUNK, CW), jnp.float32)
    cnts = _cnt(dst, ones, zrows)
    parts1 = _agg(x, src, dst, zrows)
    c0, c1 = cnts[0], cnts[1]
    h1 = _sage_tc(parts1[0], parts1[1], c0, c1, x,
                  W1_l, W1_r, b1.reshape(1, D))
    parts2 = _agg(h1, src, dst, zrows)
    return _final_tc(parts2[0], parts2[1], c0, c1, h1,
                     W2_l, W2_r, b2.reshape(1, D), Wc, bc.reshape(1, CLS))
